# Initial kernel scaffold; baseline (speedup 1.0000x reference)
import jax
import jax.numpy as jnp
from jax.experimental import pallas as pl


def kernel(x, edge_index, W1, a_src1, a_dst1, b1, W2, a_src2, a_dst2, b2):
    raise NotImplementedError("write your pallas kernel here")


# speedup vs baseline: 53.3905x; 53.3905x over previous
import functools

import jax
import jax.numpy as jnp
from jax import lax
from jax.experimental import pallas as pl
from jax.experimental.pallas import tpu as pltpu
from jax.experimental.pallas import tpu_sc as plsc

_N = 10000
_E = 320000
_IN = 128
_HID = 16
_HEADS = 8
_NC = 40

_NSC = 2
_NTILE = 16
_NW = _NSC * _NTILE
_C = 128
_NCHUNK = _E // _C
_MAXK = -(-_NCHUNK // _NW)
_NPAD = 10000
_ROWS_PER_TILE = _NPAD // _NTILE
_ZROWS = 125

_W1EXT = 144
_W2EXT = 48

_BLK = 1000


def _tc_a_body(x_ref, w1_ref, as_ref, ad_ref, h1ext_ref, adt_ref):
    h1 = jnp.dot(x_ref[...], w1_ref[...], preferred_element_type=jnp.float32)
    asrc = jnp.dot(h1, as_ref[...], preferred_element_type=jnp.float32)
    adst = jnp.dot(h1, ad_ref[...], preferred_element_type=jnp.float32)
    z8 = jnp.zeros_like(asrc)
    h1ext_ref[:, :_IN] = h1
    h1ext_ref[:, _IN:_W1EXT] = jnp.concatenate([asrc, z8], axis=1)
    adt_ref[...] = jnp.concatenate([adst, z8], axis=1)


def _tc_a(x, w1, as1, ad1):
    grid = (_N // _BLK,)
    return pl.pallas_call(
        _tc_a_body,
        grid=grid,
        in_specs=[
            pl.BlockSpec((_BLK, _IN), lambda i: (i, 0)),
            pl.BlockSpec((_IN, _IN), lambda i: (0, 0)),
            pl.BlockSpec((_IN, _HEADS), lambda i: (0, 0)),
            pl.BlockSpec((_IN, _HEADS), lambda i: (0, 0)),
        ],
        out_specs=[
            pl.BlockSpec((_BLK, _W1EXT), lambda i: (i, 0)),
            pl.BlockSpec((_BLK, 16), lambda i: (i, 0)),
        ],
        out_shape=[
            jax.ShapeDtypeStruct((_N, _W1EXT), jnp.float32),
            jax.ShapeDtypeStruct((_N, 16), jnp.float32),
        ],
    )(x, w1, as1, ad1)


def _tc_b_body(a0_ref, a1_ref, b1_ref, w2_ref, r_ref, a2s_ref, a2d_ref,
               h2ext_ref, adt2_ref):
    acc = a0_ref[...] + a1_ref[...]
    inv = 1.0 / (acc[:, _IN:_IN + _HEADS] + 1e-16)
    inv128 = jnp.dot(inv, r_ref[...], preferred_element_type=jnp.float32)
    h2in = jnp.maximum(acc[:, :_IN] * inv128 + b1_ref[...], 0.0)
    h2 = jnp.dot(h2in, w2_ref[...], preferred_element_type=jnp.float32)
    asrc2 = jnp.dot(h2, a2s_ref[...], preferred_element_type=jnp.float32)
    adst2 = jnp.dot(h2, a2d_ref[...], preferred_element_type=jnp.float32)
    ones = jnp.ones_like(asrc2)
    z6 = jnp.zeros((h2.shape[0], 6), jnp.float32)
    h2ext_ref[...] = jnp.concatenate([h2, ones, asrc2, z6], axis=1)
    adt2_ref[...] = jnp.broadcast_to(adst2, (h2.shape[0], 16))


def _tc_b(a0, a1, b1r, w2, r, a2s, a2d):
    grid = (_N // _BLK,)
    return pl.pallas_call(
        _tc_b_body,
        grid=grid,
        in_specs=[
            pl.BlockSpec((_BLK, _W1EXT), lambda i: (i, 0)),
            pl.BlockSpec((_BLK, _W1EXT), lambda i: (i, 0)),
            pl.BlockSpec((1, _IN), lambda i: (0, 0)),
            pl.BlockSpec((_IN, _NC), lambda i: (0, 0)),
            pl.BlockSpec((_HEADS, _IN), lambda i: (0, 0)),
            pl.BlockSpec((_NC, 1), lambda i: (0, 0)),
            pl.BlockSpec((_NC, 1), lambda i: (0, 0)),
        ],
        out_specs=[
            pl.BlockSpec((_BLK, _W2EXT), lambda i: (i, 0)),
            pl.BlockSpec((_BLK, 16), lambda i: (i, 0)),
        ],
        out_shape=[
            jax.ShapeDtypeStruct((_N, _W2EXT), jnp.float32),
            jax.ShapeDtypeStruct((_N, 16), jnp.float32),
        ],
    )(a0, a1, b1r, w2, r, a2s, a2d)


def _tc_c_body(a0_ref, a1_ref, b2_ref, out_ref):
    acc = a0_ref[...] + a1_ref[...]
    den = acc[:, _NC:_NC + 1] + 1e-16
    out_ref[...] = acc[:, :_NC] / den + b2_ref[...]


def _tc_c(a0, a1, b2r):
    grid = (_N // _BLK,)
    return pl.pallas_call(
        _tc_c_body,
        grid=grid,
        in_specs=[
            pl.BlockSpec((_BLK, _W2EXT), lambda i: (i, 0)),
            pl.BlockSpec((_BLK, _W2EXT), lambda i: (i, 0)),
            pl.BlockSpec((1, _NC), lambda i: (0, 0)),
        ],
        out_specs=pl.BlockSpec((_BLK, _NC), lambda i: (i, 0)),
        out_shape=jax.ShapeDtypeStruct((_N, _NC), jnp.float32),
    )(a0, a1, b2r)


def _sc_zero_acc(s, zbuf, acc, width):
    def zrow(r, carry):
        for j in range(width // 16):
            zbuf[r, pl.ds(j * 16, 16)] = jnp.zeros((16,), jnp.float32)
        return carry
    lax.fori_loop(0, _ZROWS, zrow, 0)
    for t in range(_ROWS_PER_TILE // _ZROWS):
        pltpu.sync_copy(
            zbuf, acc.at[pl.ds(s * _ROWS_PER_TILE + t * _ZROWS, _ZROWS)])


def _sc_drain(c, s, acc, out_hbm):
    for t in range(_ROWS_PER_TILE // _ZROWS):
        sl = pl.ds(s * _ROWS_PER_TILE + t * _ZROWS, _ZROWS)
        pltpu.sync_copy(acc.at[sl], out_hbm.at[c, sl])


def _sc1_body(src_hbm, dst_hbm, h1ext_hbm, adt_hbm, out_hbm,
              srcv, dstv, rows, adv, zbuf, acc, sem1, sem2):
    c = lax.axis_index("c")
    s = lax.axis_index("s")
    wid = s * _NSC + c
    _sc_zero_acc(s, zbuf, acc, _W1EXT)
    plsc.subcore_barrier()

    def chunk_body(k, carry):
        i = wid + _NW * k

        @pl.when(i < _NCHUNK)
        def _():
            base = i * _C
            pltpu.sync_copy(src_hbm.at[pl.ds(base, _C)], srcv)
            pltpu.sync_copy(dst_hbm.at[pl.ds(base, _C)], dstv)
            cp1 = pltpu.async_copy(h1ext_hbm.at[srcv], rows, sem1)
            cp2 = pltpu.async_copy(adt_hbm.at[dstv], adv, sem2)
            cp1.wait()
            cp2.wait()

            def ebody(e, ecarry):
                al = rows[e, pl.ds(_IN, 16)] + adv[e, :]
                al = jnp.where(al >= 0.0, al, al * 0.2)
                exv = jnp.exp(al)
                rows[e, pl.ds(_IN, 16)] = exv
                for h in range(_HEADS):
                    rows[e, pl.ds(h * _HID, _HID)] = (
                        rows[e, pl.ds(h * _HID, _HID)] * exv[h])
                return ecarry

            lax.fori_loop(0, _C, ebody, 0)
            pltpu.sync_copy(rows, acc.at[dstv], add=True)

        return carry

    lax.fori_loop(0, _MAXK, chunk_body, 0)
    plsc.subcore_barrier()
    _sc_drain(c, s, acc, out_hbm)


def _sc_pass1(src, dst, h1ext, adt):
    mesh = plsc.VectorSubcoreMesh(
        core_axis_name="c", subcore_axis_name="s",
        num_cores=_NSC, num_subcores=_NTILE)
    return pl.kernel(
        _sc1_body,
        compiler_params=pltpu.CompilerParams(use_tc_tiling_on_sc=False),
        out_type=jax.ShapeDtypeStruct((_NSC, _NPAD, _W1EXT), jnp.float32),
        mesh=mesh,
        scratch_types=[
            pltpu.VMEM((_C,), jnp.int32),
            pltpu.VMEM((_C,), jnp.int32),
            pltpu.VMEM((_C, _W1EXT), jnp.float32),
            pltpu.VMEM((_C, 16), jnp.float32),
            pltpu.VMEM((_ZROWS, _W1EXT), jnp.float32),
            pltpu.VMEM_SHARED((_NPAD, _W1EXT), jnp.float32),
            pltpu.SemaphoreType.DMA,
            pltpu.SemaphoreType.DMA,
        ],
    )(src, dst, h1ext, adt)


def _sc2_body(src_hbm, dst_hbm, h2ext_hbm, adt_hbm, out_hbm,
              srcv, dstv, rows, adv, zbuf, acc, sem1, sem2):
    c = lax.axis_index("c")
    s = lax.axis_index("s")
    wid = s * _NSC + c
    _sc_zero_acc(s, zbuf, acc, _W2EXT)
    plsc.subcore_barrier()

    def chunk_body(k, carry):
        i = wid + _NW * k

        @pl.when(i < _NCHUNK)
        def _():
            base = i * _C
            pltpu.sync_copy(src_hbm.at[pl.ds(base, _C)], srcv)
            pltpu.sync_copy(dst_hbm.at[pl.ds(base, _C)], dstv)
            cp1 = pltpu.async_copy(h2ext_hbm.at[srcv], rows, sem1)
            cp2 = pltpu.async_copy(adt_hbm.at[dstv], adv, sem2)
            cp1.wait()
            cp2.wait()

            def ebody(e, ecarry):
                v2 = rows[e, pl.ds(32, 16)]
                al = v2 + adv[e, :]
                al = jnp.where(al >= 0.0, al, al * 0.2)
                ex = jnp.exp(al)[9]
                rows[e, pl.ds(0, 16)] = rows[e, pl.ds(0, 16)] * ex
                rows[e, pl.ds(16, 16)] = rows[e, pl.ds(16, 16)] * ex
                rows[e, pl.ds(32, 16)] = v2 * ex
                return ecarry

            lax.fori_loop(0, _C, ebody, 0)
            pltpu.sync_copy(rows, acc.at[dstv], add=True)

        return carry

    lax.fori_loop(0, _MAXK, chunk_body, 0)
    plsc.subcore_barrier()
    _sc_drain(c, s, acc, out_hbm)


def _sc_pass2(src, dst, h2ext, adt2):
    mesh = plsc.VectorSubcoreMesh(
        core_axis_name="c", subcore_axis_name="s",
        num_cores=_NSC, num_subcores=_NTILE)
    return pl.kernel(
        _sc2_body,
        compiler_params=pltpu.CompilerParams(use_tc_tiling_on_sc=False),
        out_type=jax.ShapeDtypeStruct((_NSC, _NPAD, _W2EXT), jnp.float32),
        mesh=mesh,
        scratch_types=[
            pltpu.VMEM((_C,), jnp.int32),
            pltpu.VMEM((_C,), jnp.int32),
            pltpu.VMEM((_C, _W2EXT), jnp.float32),
            pltpu.VMEM((_C, 16), jnp.float32),
            pltpu.VMEM((_ZROWS, _W2EXT), jnp.float32),
            pltpu.VMEM_SHARED((_NPAD, _W2EXT), jnp.float32),
            pltpu.SemaphoreType.DMA,
            pltpu.SemaphoreType.DMA,
        ],
    )(src, dst, h2ext, adt2)


@jax.jit
def kernel(x, edge_index, W1, a_src1, a_dst1, b1, W2, a_src2, a_dst2, b2):
    src = edge_index[0].astype(jnp.int32)
    dst = edge_index[1].astype(jnp.int32)

    mask = (jnp.arange(_IN)[:, None] // _HID
            == jnp.arange(_HEADS)[None, :]).astype(jnp.float32)
    as1 = mask * a_src1.reshape(_IN)[:, None]
    ad1 = mask * a_dst1.reshape(_IN)[:, None]

    h1ext, adt1 = _tc_a(x, W1, as1, ad1)
    p1 = _sc_pass1(src, dst, h1ext, adt1)
    h2ext, adt2 = _tc_b(p1[0], p1[1], b1.reshape(1, _IN), W2, mask.T,
                        a_src2.reshape(_NC, 1), a_dst2.reshape(_NC, 1))
    p2 = _sc_pass2(src, dst, h2ext, adt2)
    return _tc_c(p2[0], p2[1], b2.reshape(1, _NC))


# speedup vs baseline: 58.3559x; 1.0930x over previous
import functools

import jax
import jax.numpy as jnp
from jax import lax
from jax.experimental import pallas as pl
from jax.experimental.pallas import tpu as pltpu
from jax.experimental.pallas import tpu_sc as plsc

_N = 10000
_E = 320000
_IN = 128
_HID = 16
_HEADS = 8
_NC = 40

_NSC = 2
_NTILE = 16
_NW = _NSC * _NTILE
_C = 96
_KPW = 106
_NCHUNK = _NW * _KPW
_EPAD = _NCHUNK * _C
_NTAB = _N + 16
_ROWS_PER_TILE = _N // _NTILE
_ZROWS = 25

_W1EXT = 144
_W2EXT = 48

_BLK = 1000


def _tc_a_body(x_ref, w1_ref, as_ref, ad_ref, h1ext_ref, adt_ref):
    h1 = jnp.dot(x_ref[...], w1_ref[...], preferred_element_type=jnp.float32)
    asrc = jnp.dot(h1, as_ref[...], preferred_element_type=jnp.float32)
    adst = jnp.dot(h1, ad_ref[...], preferred_element_type=jnp.float32)
    z8 = jnp.zeros_like(asrc)
    h1ext_ref[:, :_IN] = h1
    h1ext_ref[:, _IN:_W1EXT] = jnp.concatenate([asrc, z8], axis=1)
    adt_ref[...] = jnp.concatenate([adst, z8], axis=1)


def _tc_a(x, w1, as1, ad1):
    grid = (_N // _BLK,)
    return pl.pallas_call(
        _tc_a_body,
        grid=grid,
        in_specs=[
            pl.BlockSpec((_BLK, _IN), lambda i: (i, 0)),
            pl.BlockSpec((_IN, _IN), lambda i: (0, 0)),
            pl.BlockSpec((_IN, _HEADS), lambda i: (0, 0)),
            pl.BlockSpec((_IN, _HEADS), lambda i: (0, 0)),
        ],
        out_specs=[
            pl.BlockSpec((_BLK, _W1EXT), lambda i: (i, 0)),
            pl.BlockSpec((_BLK, 16), lambda i: (i, 0)),
        ],
        out_shape=[
            jax.ShapeDtypeStruct((_N, _W1EXT), jnp.float32),
            jax.ShapeDtypeStruct((_N, 16), jnp.float32),
        ],
    )(x, w1, as1, ad1)


def _tc_b_body(a0_ref, a1_ref, b1_ref, w2_ref, r_ref, a2s_ref, a2d_ref,
               h2ext_ref, adt2_ref):
    acc = a0_ref[...] + a1_ref[...]
    inv = 1.0 / (acc[:, _IN:_IN + _HEADS] + 1e-16)
    inv128 = jnp.dot(inv, r_ref[...], preferred_element_type=jnp.float32)
    h2in = jnp.maximum(acc[:, :_IN] * inv128 + b1_ref[...], 0.0)
    h2 = jnp.dot(h2in, w2_ref[...], preferred_element_type=jnp.float32)
    asrc2 = jnp.dot(h2, a2s_ref[...], preferred_element_type=jnp.float32)
    adst2 = jnp.dot(h2, a2d_ref[...], preferred_element_type=jnp.float32)
    ones = jnp.ones_like(asrc2)
    z6 = jnp.zeros((h2.shape[0], 6), jnp.float32)
    h2ext_ref[...] = jnp.concatenate([h2, ones, asrc2, z6], axis=1)
    adt2_ref[...] = jnp.broadcast_to(adst2, (h2.shape[0], 16))


def _tc_b(a0, a1, b1r, w2, r, a2s, a2d):
    grid = (_N // _BLK,)
    return pl.pallas_call(
        _tc_b_body,
        grid=grid,
        in_specs=[
            pl.BlockSpec((_BLK, _W1EXT), lambda i: (i, 0)),
            pl.BlockSpec((_BLK, _W1EXT), lambda i: (i, 0)),
            pl.BlockSpec((1, _IN), lambda i: (0, 0)),
            pl.BlockSpec((_IN, _NC), lambda i: (0, 0)),
            pl.BlockSpec((_HEADS, _IN), lambda i: (0, 0)),
            pl.BlockSpec((_NC, 1), lambda i: (0, 0)),
            pl.BlockSpec((_NC, 1), lambda i: (0, 0)),
        ],
        out_specs=[
            pl.BlockSpec((_BLK, _W2EXT), lambda i: (i, 0)),
            pl.BlockSpec((_BLK, 16), lambda i: (i, 0)),
        ],
        out_shape=[
            jax.ShapeDtypeStruct((_N, _W2EXT), jnp.float32),
            jax.ShapeDtypeStruct((_N, 16), jnp.float32),
        ],
    )(a0, a1, b1r, w2, r, a2s, a2d)


def _tc_c_body(a0_ref, a1_ref, b2_ref, out_ref):
    acc = a0_ref[...] + a1_ref[...]
    den = acc[:, _NC:_NC + 1] + 1e-16
    out_ref[...] = acc[:, :_NC] / den + b2_ref[...]


def _tc_c(a0, a1, b2r):
    grid = (_N // _BLK,)
    return pl.pallas_call(
        _tc_c_body,
        grid=grid,
        in_specs=[
            pl.BlockSpec((_BLK, _W2EXT), lambda i: (i, 0)),
            pl.BlockSpec((_BLK, _W2EXT), lambda i: (i, 0)),
            pl.BlockSpec((1, _NC), lambda i: (0, 0)),
        ],
        out_specs=pl.BlockSpec((_BLK, _NC), lambda i: (i, 0)),
        out_shape=jax.ShapeDtypeStruct((_N, _NC), jnp.float32),
    )(a0, a1, b2r)


def _sc_zero_acc(s, zbuf, acc, width):
    def zrow(r, carry):
        for j in range(width // 16):
            zbuf[r, pl.ds(j * 16, 16)] = jnp.zeros((16,), jnp.float32)
        return carry
    lax.fori_loop(0, _ZROWS, zrow, 0)

    def zcopy(t, carry):
        pltpu.sync_copy(
            zbuf, acc.at[pl.ds(s * _ROWS_PER_TILE + t * _ZROWS, _ZROWS)])
        return carry
    lax.fori_loop(0, _ROWS_PER_TILE // _ZROWS, zcopy, 0)


def _sc_drain(c, s, acc, out_hbm):
    sl = pl.ds(s * _ROWS_PER_TILE, _ROWS_PER_TILE)
    pltpu.sync_copy(acc.at[sl], out_hbm.at[c, sl])


def _compute1(rows, adv):
    def ebody(e, ecarry):
        al = rows[e, pl.ds(_IN, 16)] + adv[e, :]
        al = jnp.where(al >= 0.0, al, al * 0.2)
        exv = jnp.exp(al)
        rows[e, pl.ds(_IN, 16)] = exv
        for h in range(_HEADS):
            rows[e, pl.ds(h * _HID, _HID)] = (
                rows[e, pl.ds(h * _HID, _HID)] * exv[h])
        return ecarry

    lax.fori_loop(0, _C, ebody, 0, unroll=4)


def _compute2(rows, adv):
    def ebody(e, ecarry):
        v2 = rows[e, pl.ds(32, 16)]
        al = v2 + adv[e, :]
        al = jnp.where(al >= 0.0, al, al * 0.2)
        ex = jnp.exp(al)[9]
        rows[e, pl.ds(0, 16)] = rows[e, pl.ds(0, 16)] * ex
        rows[e, pl.ds(16, 16)] = rows[e, pl.ds(16, 16)] * ex
        rows[e, pl.ds(32, 16)] = v2 * ex
        return ecarry

    lax.fori_loop(0, _C, ebody, 0, unroll=8)


def _make_sc_body(width, compute_chunk):
    def body(src_hbm, dst_hbm, tab_hbm, adt_hbm, out_hbm,
             srcv0, srcv1, dstv0, dstv1, dsts0, dsts1,
             rows0, rows1, adv0, adv1, zbuf, acc,
             semis0, semis1, semid0, semid1,
             semr0, semr1, sema0, sema1, sems0, sems1):
        c = lax.axis_index("c")
        s = lax.axis_index("s")
        wid = s * _NSC + c
        srcv = (srcv0, srcv1)
        dstv = (dstv0, dstv1)
        dsts = (dsts0, dsts1)
        rows = (rows0, rows1)
        adv = (adv0, adv1)
        semis = (semis0, semis1)
        semid = (semid0, semid1)
        semr = (semr0, semr1)
        sema = (sema0, sema1)
        sems = (sems0, sems1)
        start = wid * _KPW

        def idx_start(j, b):
            pltpu.async_copy(src_hbm.at[start + j], srcv[b], semis[b])
            pltpu.async_copy(dst_hbm.at[start + j], dstv[b], semid[b])

        def idx_wait(j, b):
            pltpu.make_async_copy(
                src_hbm.at[start + j], srcv[b], semis[b]).wait()
            pltpu.make_async_copy(
                dst_hbm.at[start + j], dstv[b], semid[b]).wait()

        def gather_start(b):
            pltpu.async_copy(tab_hbm.at[srcv[b]], rows[b], semr[b])
            pltpu.async_copy(adt_hbm.at[dstv[b]], adv[b], sema[b])

        def gather_wait(b):
            pltpu.make_async_copy(tab_hbm.at[srcv[b]], rows[b],
                                  semr[b]).wait()
            pltpu.make_async_copy(adt_hbm.at[dstv[b]], adv[b],
                                  sema[b]).wait()

        def scatter_wait(b):
            pltpu.make_async_copy(rows[b], acc.at[dsts[b]], sems[b]).wait()

        idx_start(0, 0)
        idx_start(1, 1)
        _sc_zero_acc(s, zbuf, acc, width)
        idx_wait(0, 0)
        gather_start(0)
        plsc.subcore_barrier()

        def outer(k, carry):
            for b in range(2):
                j = 2 * k + b
                q = 1 - b

                @pl.when(j >= 1)
                def _():
                    scatter_wait(q)

                @pl.when(j + 1 < _KPW)
                def _():
                    idx_wait(j + 1, q)
                    gather_start(q)

                gather_wait(b)
                for v in range(_C // 16):
                    dsts[b][pl.ds(v * 16, 16)] = dstv[b][pl.ds(v * 16, 16)]

                @pl.when(j + 2 < _KPW)
                def _():
                    idx_start(j + 2, b)

                compute_chunk(rows[b], adv[b])
                pltpu.async_copy(rows[b], acc.at[dsts[b]], sems[b], add=True)
            return carry

        lax.fori_loop(0, _KPW // 2, outer, 0)
        scatter_wait(1)
        plsc.subcore_barrier()
        _sc_drain(c, s, acc, out_hbm)

    return body


def _sc_pass(src2d, dst2d, tab, adt, width, compute_chunk):
    mesh = plsc.VectorSubcoreMesh(
        core_axis_name="c", subcore_axis_name="s",
        num_cores=_NSC, num_subcores=_NTILE)
    return pl.kernel(
        _make_sc_body(width, compute_chunk),
        compiler_params=pltpu.CompilerParams(use_tc_tiling_on_sc=False),
        out_type=jax.ShapeDtypeStruct((_NSC, _N, width), jnp.float32),
        mesh=mesh,
        scratch_types=(
            [pltpu.VMEM((_C,), jnp.int32) for _ in range(6)]
            + [
                pltpu.VMEM((_C, width), jnp.float32),
                pltpu.VMEM((_C, width), jnp.float32),
                pltpu.VMEM((_C, 16), jnp.float32),
                pltpu.VMEM((_C, 16), jnp.float32),
                pltpu.VMEM((_ZROWS, width), jnp.float32),
                pltpu.VMEM_SHARED((_N, width), jnp.float32),
            ]
            + [pltpu.SemaphoreType.DMA for _ in range(10)]
        ),
    )(src2d, dst2d, tab, adt)


@jax.jit
def kernel(x, edge_index, W1, a_src1, a_dst1, b1, W2, a_src2, a_dst2, b2):
    src = edge_index[0].astype(jnp.int32)
    dst = edge_index[1].astype(jnp.int32)

    mask = (jnp.arange(_IN)[:, None] // _HID
            == jnp.arange(_HEADS)[None, :]).astype(jnp.float32)
    as1 = mask * a_src1.reshape(_IN)[:, None]
    ad1 = mask * a_dst1.reshape(_IN)[:, None]

    npad = _EPAD - _E
    src2d = jnp.pad(src, (0, npad), constant_values=_N).reshape(_NCHUNK, _C)
    dst2d = jnp.pad(dst, (0, npad)).reshape(_NCHUNK, _C)
    ninf = jnp.float32(-jnp.inf)
    pad1 = jnp.concatenate(
        [jnp.zeros((_NTAB - _N, _IN), jnp.float32),
         jnp.full((_NTAB - _N, 16), ninf)], axis=1)
    pad2 = jnp.concatenate(
        [jnp.zeros((_NTAB - _N, _NC + 1), jnp.float32),
         jnp.full((_NTAB - _N, 1), ninf),
         jnp.zeros((_NTAB - _N, 6), jnp.float32)], axis=1)

    h1ext, adt1 = _tc_a(x, W1, as1, ad1)
    p1 = _sc_pass(src2d, dst2d, jnp.concatenate([h1ext, pad1]), adt1,
                  _W1EXT, _compute1)
    h2ext, adt2 = _tc_b(p1[0], p1[1], b1.reshape(1, _IN), W2, mask.T,
                        a_src2.reshape(_NC, 1), a_dst2.reshape(_NC, 1))
    p2 = _sc_pass(src2d, dst2d, jnp.concatenate([h2ext, pad2]), adt2,
                  _W2EXT, _compute2)
    return _tc_c(p2[0], p2[1], b2.reshape(1, _NC))


# speedup vs baseline: 75.7464x; 1.2980x over previous
import functools

import jax
import jax.numpy as jnp
from jax import lax
from jax.experimental import pallas as pl
from jax.experimental.pallas import tpu as pltpu
from jax.experimental.pallas import tpu_sc as plsc

_N = 10000
_E = 320000
_IN = 128
_HID = 16
_HEADS = 8
_NC = 40

_NSC = 2
_NTILE = 16
_NW = _NSC * _NTILE
_C = 96
_KPW = 106
_NCHUNK = _NW * _KPW
_EPAD = _NCHUNK * _C
_NTAB = _N + 16
_ROWS_PER_TILE = _N // _NTILE
_ZROWS = 25

_W1EXT = 144
_W2EXT = 48

_BLK = 1000


def _tc_a_body(x_ref, w1_ref, as_ref, ad_ref, h1ext_ref, adt_ref):
    h1 = jnp.dot(x_ref[...], w1_ref[...], preferred_element_type=jnp.float32)
    asrc = jnp.dot(h1, as_ref[...], preferred_element_type=jnp.float32)
    adst = jnp.dot(h1, ad_ref[...], preferred_element_type=jnp.float32)
    z8 = jnp.zeros_like(asrc)
    h1ext_ref[:, :_IN] = h1
    h1ext_ref[:, _IN:_W1EXT] = jnp.concatenate([asrc, z8], axis=1)
    adt_ref[...] = jnp.concatenate([adst, z8], axis=1)


def _tc_a(x, w1, as1, ad1):
    grid = (_N // _BLK,)
    return pl.pallas_call(
        _tc_a_body,
        grid=grid,
        in_specs=[
            pl.BlockSpec((_BLK, _IN), lambda i: (i, 0)),
            pl.BlockSpec((_IN, _IN), lambda i: (0, 0)),
            pl.BlockSpec((_IN, _HEADS), lambda i: (0, 0)),
            pl.BlockSpec((_IN, _HEADS), lambda i: (0, 0)),
        ],
        out_specs=[
            pl.BlockSpec((_BLK, _W1EXT), lambda i: (i, 0)),
            pl.BlockSpec((_BLK, 16), lambda i: (i, 0)),
        ],
        out_shape=[
            jax.ShapeDtypeStruct((_N, _W1EXT), jnp.float32),
            jax.ShapeDtypeStruct((_N, 16), jnp.float32),
        ],
    )(x, w1, as1, ad1)


def _tc_b_body(a0_ref, a1_ref, b1_ref, w2_ref, r_ref, a2s_ref, a2d_ref,
               h2ext_ref, adt2_ref):
    acc = a0_ref[...] + a1_ref[...]
    inv = 1.0 / (acc[:, _IN:_IN + _HEADS] + 1e-16)
    inv128 = jnp.dot(inv, r_ref[...], preferred_element_type=jnp.float32)
    h2in = jnp.maximum(acc[:, :_IN] * inv128 + b1_ref[...], 0.0)
    h2 = jnp.dot(h2in, w2_ref[...], preferred_element_type=jnp.float32)
    asrc2 = jnp.dot(h2, a2s_ref[...], preferred_element_type=jnp.float32)
    adst2 = jnp.dot(h2, a2d_ref[...], preferred_element_type=jnp.float32)
    ones = jnp.ones_like(asrc2)
    z6 = jnp.zeros((h2.shape[0], 6), jnp.float32)
    h2ext_ref[...] = jnp.concatenate([h2, ones, asrc2, z6], axis=1)
    adt2_ref[...] = jnp.broadcast_to(adst2, (h2.shape[0], 16))


def _tc_b(a0, a1, b1r, w2, r, a2s, a2d):
    grid = (_N // _BLK,)
    return pl.pallas_call(
        _tc_b_body,
        grid=grid,
        in_specs=[
            pl.BlockSpec((_BLK, _W1EXT), lambda i: (i, 0)),
            pl.BlockSpec((_BLK, _W1EXT), lambda i: (i, 0)),
            pl.BlockSpec((1, _IN), lambda i: (0, 0)),
            pl.BlockSpec((_IN, _NC), lambda i: (0, 0)),
            pl.BlockSpec((_HEADS, _IN), lambda i: (0, 0)),
            pl.BlockSpec((_NC, 1), lambda i: (0, 0)),
            pl.BlockSpec((_NC, 1), lambda i: (0, 0)),
        ],
        out_specs=[
            pl.BlockSpec((_BLK, _W2EXT), lambda i: (i, 0)),
            pl.BlockSpec((_BLK, 16), lambda i: (i, 0)),
        ],
        out_shape=[
            jax.ShapeDtypeStruct((_N, _W2EXT), jnp.float32),
            jax.ShapeDtypeStruct((_N, 16), jnp.float32),
        ],
    )(a0, a1, b1r, w2, r, a2s, a2d)


def _tc_c_body(a0_ref, a1_ref, b2_ref, out_ref):
    acc = a0_ref[...] + a1_ref[...]
    den = acc[:, _NC:_NC + 1] + 1e-16
    out_ref[...] = acc[:, :_NC] / den + b2_ref[...]


def _tc_c(a0, a1, b2r):
    grid = (_N // _BLK,)
    return pl.pallas_call(
        _tc_c_body,
        grid=grid,
        in_specs=[
            pl.BlockSpec((_BLK, _W2EXT), lambda i: (i, 0)),
            pl.BlockSpec((_BLK, _W2EXT), lambda i: (i, 0)),
            pl.BlockSpec((1, _NC), lambda i: (0, 0)),
        ],
        out_specs=pl.BlockSpec((_BLK, _NC), lambda i: (i, 0)),
        out_shape=jax.ShapeDtypeStruct((_N, _NC), jnp.float32),
    )(a0, a1, b2r)


def _sc_zero_acc(s, zbuf, acc, width):
    def zrow(r, carry):
        for j in range(width // 16):
            zbuf[r, pl.ds(j * 16, 16)] = jnp.zeros((16,), jnp.float32)
        return carry
    lax.fori_loop(0, _ZROWS, zrow, 0)

    def zcopy(t, carry):
        pltpu.sync_copy(
            zbuf, acc.at[pl.ds(s * _ROWS_PER_TILE + t * _ZROWS, _ZROWS)])
        return carry
    lax.fori_loop(0, _ROWS_PER_TILE // _ZROWS, zcopy, 0)


def _sc_drain(c, s, acc, out_hbm):
    sl = pl.ds(s * _ROWS_PER_TILE, _ROWS_PER_TILE)
    pltpu.sync_copy(acc.at[sl], out_hbm.at[c, sl])


def _compute1(rows, adv):
    def ebody(e, ecarry):
        al = rows[e, pl.ds(_IN, 16)] + adv[e, :]
        al = jnp.where(al >= 0.0, al, al * 0.2)
        exv = jnp.exp(al)
        rows[e, pl.ds(_IN, 16)] = exv
        for h in range(_HEADS):
            rows[e, pl.ds(h * _HID, _HID)] = (
                rows[e, pl.ds(h * _HID, _HID)] * exv[h])
        return ecarry

    lax.fori_loop(0, _C, ebody, 0, unroll=4)


def _compute2(rows, adv):
    def ebody(e, ecarry):
        v2 = rows[e, pl.ds(32, 16)]
        al = v2 + adv[e, :]
        al = jnp.where(al >= 0.0, al, al * 0.2)
        ex = jnp.exp(al)[9]
        rows[e, pl.ds(0, 16)] = rows[e, pl.ds(0, 16)] * ex
        rows[e, pl.ds(16, 16)] = rows[e, pl.ds(16, 16)] * ex
        rows[e, pl.ds(32, 16)] = v2 * ex
        return ecarry

    lax.fori_loop(0, _C, ebody, 0, unroll=8)


def _make_sc_body(width, compute_chunk):
    def body(src_hbm, dst_hbm, tab_hbm, adt_hbm, out_hbm,
             srcv0, srcv1, dstv0, dstv1, dsts0, dsts1,
             rows0, rows1, adv0, adv1, zbuf, acc,
             semis0, semis1, semid0, semid1,
             semr0, semr1, sema0, sema1, sems0, sems1):
        c = lax.axis_index("c")
        s = lax.axis_index("s")
        wid = s * _NSC + c
        srcv = (srcv0, srcv1)
        dstv = (dstv0, dstv1)
        dsts = (dsts0, dsts1)
        rows = (rows0, rows1)
        adv = (adv0, adv1)
        semis = (semis0, semis1)
        semid = (semid0, semid1)
        semr = (semr0, semr1)
        sema = (sema0, sema1)
        sems = (sems0, sems1)
        start = wid * _KPW

        def idx_start(j, b):
            pltpu.async_copy(src_hbm.at[start + j], srcv[b], semis[b])
            pltpu.async_copy(dst_hbm.at[start + j], dstv[b], semid[b])

        def idx_wait(j, b):
            pltpu.make_async_copy(
                src_hbm.at[start + j], srcv[b], semis[b]).wait()
            pltpu.make_async_copy(
                dst_hbm.at[start + j], dstv[b], semid[b]).wait()

        def gather_start(b):
            pltpu.async_copy(tab_hbm.at[srcv[b]], rows[b], semr[b])
            pltpu.async_copy(adt_hbm.at[dstv[b]], adv[b], sema[b])

        def gather_wait(b):
            pltpu.make_async_copy(tab_hbm.at[srcv[b]], rows[b],
                                  semr[b]).wait()
            pltpu.make_async_copy(adt_hbm.at[dstv[b]], adv[b],
                                  sema[b]).wait()

        def scatter_wait(b):
            pltpu.make_async_copy(rows[b], acc.at[dsts[b]], sems[b]).wait()

        idx_start(0, 0)
        idx_start(1, 1)
        _sc_zero_acc(s, zbuf, acc, width)
        idx_wait(0, 0)
        gather_start(0)
        plsc.subcore_barrier()

        def outer(k, carry):
            for b in range(2):
                j = 2 * k + b
                q = 1 - b

                @pl.when(j >= 1)
                def _():
                    scatter_wait(q)

                @pl.when(j + 1 < _KPW)
                def _():
                    idx_wait(j + 1, q)
                    gather_start(q)

                gather_wait(b)
                for v in range(_C // 16):
                    dsts[b][pl.ds(v * 16, 16)] = dstv[b][pl.ds(v * 16, 16)]

                @pl.when(j + 2 < _KPW)
                def _():
                    idx_start(j + 2, b)

                compute_chunk(rows[b], adv[b])
                pltpu.async_copy(rows[b], acc.at[dsts[b]], sems[b], add=True)
            return carry

        lax.fori_loop(0, _KPW // 2, outer, 0)
        scatter_wait(1)
        plsc.subcore_barrier()
        _sc_drain(c, s, acc, out_hbm)

    return body


def _sc_pass(src2d, dst2d, tab, adt, width, compute_chunk):
    mesh = plsc.VectorSubcoreMesh(
        core_axis_name="c", subcore_axis_name="s",
        num_cores=_NSC, num_subcores=_NTILE)
    return pl.kernel(
        _make_sc_body(width, compute_chunk),
        compiler_params=pltpu.CompilerParams(use_tc_tiling_on_sc=False),
        out_type=jax.ShapeDtypeStruct((_NSC, _N, width), jnp.float32),
        mesh=mesh,
        scratch_types=(
            [pltpu.VMEM((_C,), jnp.int32) for _ in range(6)]
            + [
                pltpu.VMEM((_C, width), jnp.float32),
                pltpu.VMEM((_C, width), jnp.float32),
                pltpu.VMEM((_C, 16), jnp.float32),
                pltpu.VMEM((_C, 16), jnp.float32),
                pltpu.VMEM((_ZROWS, width), jnp.float32),
                pltpu.VMEM_SHARED((_N, width), jnp.float32),
            ]
            + [pltpu.SemaphoreType.DMA for _ in range(10)]
        ),
    )(src2d, dst2d, tab, adt)


@jax.jit
def kernel(x, edge_index, W1, a_src1, a_dst1, b1, W2, a_src2, a_dst2, b2):
    src = edge_index[0].astype(jnp.int32)
    dst = edge_index[1].astype(jnp.int32)

    mask = (jnp.arange(_IN)[:, None] // _HID
            == jnp.arange(_HEADS)[None, :]).astype(jnp.float32)
    as1 = mask * a_src1.reshape(_IN)[:, None]
    ad1 = mask * a_dst1.reshape(_IN)[:, None]

    epw = _E // _NW
    ppw = _KPW * _C - epw
    pad_src = jnp.broadcast_to(
        _N + (jnp.arange(ppw, dtype=jnp.int32) % 16), (_NW, ppw))
    pad_dst = jnp.broadcast_to(
        (jnp.arange(ppw, dtype=jnp.int32) * 53) % _N, (_NW, ppw))
    src2d = jnp.concatenate(
        [src.reshape(_NW, epw), pad_src], axis=1).reshape(_NCHUNK, _C)
    dst2d = jnp.concatenate(
        [dst.reshape(_NW, epw), pad_dst], axis=1).reshape(_NCHUNK, _C)
    ninf = jnp.float32(-jnp.inf)
    pad1 = jnp.concatenate(
        [jnp.zeros((_NTAB - _N, _IN), jnp.float32),
         jnp.full((_NTAB - _N, 16), ninf)], axis=1)
    pad2 = jnp.concatenate(
        [jnp.zeros((_NTAB - _N, _NC + 1), jnp.float32),
         jnp.full((_NTAB - _N, 1), ninf),
         jnp.zeros((_NTAB - _N, 6), jnp.float32)], axis=1)

    h1ext, adt1 = _tc_a(x, W1, as1, ad1)
    p1 = _sc_pass(src2d, dst2d, jnp.concatenate([h1ext, pad1]), adt1,
                  _W1EXT, _compute1)
    h2ext, adt2 = _tc_b(p1[0], p1[1], b1.reshape(1, _IN), W2, mask.T,
                        a_src2.reshape(_NC, 1), a_dst2.reshape(_NC, 1))
    p2 = _sc_pass(src2d, dst2d, jnp.concatenate([h2ext, pad2]), adt2,
                  _W2EXT, _compute2)
    return _tc_c(p2[0], p2[1], b2.reshape(1, _NC))


# speedup vs baseline: 118.1010x; 1.5592x over previous
import functools

import jax
import jax.numpy as jnp
from jax import lax
from jax.experimental import pallas as pl
from jax.experimental.pallas import tpu as pltpu
from jax.experimental.pallas import tpu_sc as plsc

_N = 10000
_E = 320000
_IN = 128
_HID = 16
_HEADS = 8
_NC = 40

_NSC = 2
_NTILE = 16
_NW = _NSC * _NTILE
_C = 96
_KPW = 106
_NCHUNK = _NW * _KPW
_EPAD = _NCHUNK * _C
_NTAB = _N + 16
_ROWS_PER_TILE = _N // _NTILE
_ZROWS = 25

_W1EXT = 144
_W2EXT = 48

_BLK = 1000


def _tc_a_body(x_ref, w1_ref, as_ref, ad_ref, h1ext_ref, adt_ref):
    h1 = jnp.dot(x_ref[...], w1_ref[...], preferred_element_type=jnp.float32)
    asrc = jnp.dot(h1, as_ref[...], preferred_element_type=jnp.float32)
    adst = jnp.dot(h1, ad_ref[...], preferred_element_type=jnp.float32)
    z8 = jnp.zeros_like(asrc)
    h1ext_ref[:, :_IN] = h1
    h1ext_ref[:, _IN:_W1EXT] = jnp.concatenate([asrc, z8], axis=1)
    adt_ref[...] = jnp.concatenate([adst, z8], axis=1)


def _tc_a(x, w1, as1, ad1):
    grid = (_N // _BLK,)
    return pl.pallas_call(
        _tc_a_body,
        grid=grid,
        in_specs=[
            pl.BlockSpec((_BLK, _IN), lambda i: (i, 0)),
            pl.BlockSpec((_IN, _IN), lambda i: (0, 0)),
            pl.BlockSpec((_IN, _HEADS), lambda i: (0, 0)),
            pl.BlockSpec((_IN, _HEADS), lambda i: (0, 0)),
        ],
        out_specs=[
            pl.BlockSpec((_BLK, _W1EXT), lambda i: (i, 0)),
            pl.BlockSpec((_BLK, 16), lambda i: (i, 0)),
        ],
        out_shape=[
            jax.ShapeDtypeStruct((_N, _W1EXT), jnp.float32),
            jax.ShapeDtypeStruct((_N, 16), jnp.float32),
        ],
    )(x, w1, as1, ad1)


def _tc_b_body(a0_ref, a1_ref, b1_ref, w2_ref, r_ref, a2s_ref, a2d_ref,
               h2ext_ref, adt2_ref):
    acc = a0_ref[...] + a1_ref[...]
    inv = 1.0 / (acc[:, _IN:_IN + _HEADS] + 1e-16)
    inv128 = jnp.dot(inv, r_ref[...], preferred_element_type=jnp.float32)
    h2in = jnp.maximum(acc[:, :_IN] * inv128 + b1_ref[...], 0.0)
    h2 = jnp.dot(h2in, w2_ref[...], preferred_element_type=jnp.float32)
    asrc2 = jnp.dot(h2, a2s_ref[...], preferred_element_type=jnp.float32)
    adst2 = jnp.dot(h2, a2d_ref[...], preferred_element_type=jnp.float32)
    ones = jnp.ones_like(asrc2)
    z6 = jnp.zeros((h2.shape[0], 6), jnp.float32)
    h2ext_ref[...] = jnp.concatenate([h2, ones, asrc2, z6], axis=1)
    adt2_ref[...] = jnp.broadcast_to(adst2, (h2.shape[0], 16))


def _tc_b(a0, a1, b1r, w2, r, a2s, a2d):
    grid = (_N // _BLK,)
    return pl.pallas_call(
        _tc_b_body,
        grid=grid,
        in_specs=[
            pl.BlockSpec((_BLK, _W1EXT), lambda i: (i, 0)),
            pl.BlockSpec((_BLK, _W1EXT), lambda i: (i, 0)),
            pl.BlockSpec((1, _IN), lambda i: (0, 0)),
            pl.BlockSpec((_IN, _NC), lambda i: (0, 0)),
            pl.BlockSpec((_HEADS, _IN), lambda i: (0, 0)),
            pl.BlockSpec((_NC, 1), lambda i: (0, 0)),
            pl.BlockSpec((_NC, 1), lambda i: (0, 0)),
        ],
        out_specs=[
            pl.BlockSpec((_BLK, _W2EXT), lambda i: (i, 0)),
            pl.BlockSpec((_BLK, 16), lambda i: (i, 0)),
        ],
        out_shape=[
            jax.ShapeDtypeStruct((_N, _W2EXT), jnp.float32),
            jax.ShapeDtypeStruct((_N, 16), jnp.float32),
        ],
    )(a0, a1, b1r, w2, r, a2s, a2d)


def _tc_c_body(a0_ref, a1_ref, b2_ref, out_ref):
    acc = a0_ref[...] + a1_ref[...]
    den = acc[:, _NC:_NC + 1] + 1e-16
    out_ref[...] = acc[:, :_NC] / den + b2_ref[...]


def _tc_c(a0, a1, b2r):
    grid = (_N // _BLK,)
    return pl.pallas_call(
        _tc_c_body,
        grid=grid,
        in_specs=[
            pl.BlockSpec((_BLK, _W2EXT), lambda i: (i, 0)),
            pl.BlockSpec((_BLK, _W2EXT), lambda i: (i, 0)),
            pl.BlockSpec((1, _NC), lambda i: (0, 0)),
        ],
        out_specs=pl.BlockSpec((_BLK, _NC), lambda i: (i, 0)),
        out_shape=jax.ShapeDtypeStruct((_N, _NC), jnp.float32),
    )(a0, a1, b2r)


def _sc_zero_acc(s, zbuf, acc, width):
    def zrow(r, carry):
        for j in range(width // 16):
            zbuf[r, pl.ds(j * 16, 16)] = jnp.zeros((16,), jnp.float32)
        return carry
    lax.fori_loop(0, _ZROWS, zrow, 0)

    def zcopy(t, carry):
        pltpu.sync_copy(
            zbuf, acc.at[pl.ds(s * _ROWS_PER_TILE + t * _ZROWS, _ZROWS)])
        return carry
    lax.fori_loop(0, _ROWS_PER_TILE // _ZROWS, zcopy, 0)


def _sc_drain(c, s, acc, out_hbm):
    sl = pl.ds(s * _ROWS_PER_TILE, _ROWS_PER_TILE)
    pltpu.sync_copy(acc.at[sl], out_hbm.at[c, sl])


def _compute1(rows, adv):
    @plsc.parallel_loop(0, _C, unroll=4)
    def ebody(e):
        al = rows[e, pl.ds(_IN, 16)] + adv[e, :]
        al = jnp.where(al >= 0.0, al, al * 0.2)
        exv = jnp.exp(al)
        rows[e, pl.ds(_IN, 16)] = exv
        for h in range(_HEADS):
            rows[e, pl.ds(h * _HID, _HID)] = (
                rows[e, pl.ds(h * _HID, _HID)] * exv[h])


def _compute2(rows, adv):
    @plsc.parallel_loop(0, _C, unroll=8)
    def ebody(e):
        v2 = rows[e, pl.ds(32, 16)]
        al = v2 + adv[e, :]
        al = jnp.where(al >= 0.0, al, al * 0.2)
        ex = jnp.exp(al)[9]
        rows[e, pl.ds(0, 16)] = rows[e, pl.ds(0, 16)] * ex
        rows[e, pl.ds(16, 16)] = rows[e, pl.ds(16, 16)] * ex
        rows[e, pl.ds(32, 16)] = v2 * ex


def _make_sc_body(width, compute_chunk):
    def body(src_hbm, dst_hbm, tab_hbm, adt_hbm, out_hbm,
             srcv0, srcv1, dstv0, dstv1, dsts0, dsts1,
             rows0, rows1, adv0, adv1, zbuf, acc,
             semis0, semis1, semid0, semid1,
             semr0, semr1, sema0, sema1, sems0, sems1):
        c = lax.axis_index("c")
        s = lax.axis_index("s")
        wid = s * _NSC + c
        srcv = (srcv0, srcv1)
        dstv = (dstv0, dstv1)
        dsts = (dsts0, dsts1)
        rows = (rows0, rows1)
        adv = (adv0, adv1)
        semis = (semis0, semis1)
        semid = (semid0, semid1)
        semr = (semr0, semr1)
        sema = (sema0, sema1)
        sems = (sems0, sems1)
        start = wid * _KPW

        def idx_start(j, b):
            pltpu.async_copy(src_hbm.at[start + j], srcv[b], semis[b])
            pltpu.async_copy(dst_hbm.at[start + j], dstv[b], semid[b])

        def idx_wait(j, b):
            pltpu.make_async_copy(
                src_hbm.at[start + j], srcv[b], semis[b]).wait()
            pltpu.make_async_copy(
                dst_hbm.at[start + j], dstv[b], semid[b]).wait()

        def gather_start(b):
            pltpu.async_copy(tab_hbm.at[srcv[b]], rows[b], semr[b])
            pltpu.async_copy(adt_hbm.at[dstv[b]], adv[b], sema[b])

        def gather_wait(b):
            pltpu.make_async_copy(tab_hbm.at[srcv[b]], rows[b],
                                  semr[b]).wait()
            pltpu.make_async_copy(adt_hbm.at[dstv[b]], adv[b],
                                  sema[b]).wait()

        def scatter_wait(b):
            pltpu.make_async_copy(rows[b], acc.at[dsts[b]], sems[b]).wait()

        idx_start(0, 0)
        idx_start(1, 1)
        _sc_zero_acc(s, zbuf, acc, width)
        idx_wait(0, 0)
        gather_start(0)
        plsc.subcore_barrier()

        def outer(k, carry):
            for b in range(2):
                j = 2 * k + b
                q = 1 - b

                @pl.when(j >= 1)
                def _():
                    scatter_wait(q)

                @pl.when(j + 1 < _KPW)
                def _():
                    idx_wait(j + 1, q)
                    gather_start(q)

                gather_wait(b)
                for v in range(_C // 16):
                    dsts[b][pl.ds(v * 16, 16)] = dstv[b][pl.ds(v * 16, 16)]

                @pl.when(j + 2 < _KPW)
                def _():
                    idx_start(j + 2, b)

                compute_chunk(rows[b], adv[b])
                pltpu.async_copy(rows[b], acc.at[dsts[b]], sems[b], add=True)
            return carry

        lax.fori_loop(0, _KPW // 2, outer, 0)
        scatter_wait(1)
        plsc.subcore_barrier()
        _sc_drain(c, s, acc, out_hbm)

    return body


def _sc_pass(src2d, dst2d, tab, adt, width, compute_chunk):
    mesh = plsc.VectorSubcoreMesh(
        core_axis_name="c", subcore_axis_name="s",
        num_cores=_NSC, num_subcores=_NTILE)
    return pl.kernel(
        _make_sc_body(width, compute_chunk),
        compiler_params=pltpu.CompilerParams(use_tc_tiling_on_sc=False),
        out_type=jax.ShapeDtypeStruct((_NSC, _N, width), jnp.float32),
        mesh=mesh,
        scratch_types=(
            [pltpu.VMEM((_C,), jnp.int32) for _ in range(6)]
            + [
                pltpu.VMEM((_C, width), jnp.float32),
                pltpu.VMEM((_C, width), jnp.float32),
                pltpu.VMEM((_C, 16), jnp.float32),
                pltpu.VMEM((_C, 16), jnp.float32),
                pltpu.VMEM((_ZROWS, width), jnp.float32),
                pltpu.VMEM_SHARED((_N, width), jnp.float32),
            ]
            + [pltpu.SemaphoreType.DMA for _ in range(10)]
        ),
    )(src2d, dst2d, tab, adt)


@jax.jit
def kernel(x, edge_index, W1, a_src1, a_dst1, b1, W2, a_src2, a_dst2, b2):
    src = edge_index[0].astype(jnp.int32)
    dst = edge_index[1].astype(jnp.int32)

    mask = (jnp.arange(_IN)[:, None] // _HID
            == jnp.arange(_HEADS)[None, :]).astype(jnp.float32)
    as1 = mask * a_src1.reshape(_IN)[:, None]
    ad1 = mask * a_dst1.reshape(_IN)[:, None]

    epw = _E // _NW
    ppw = _KPW * _C - epw
    pad_src = jnp.broadcast_to(
        _N + (jnp.arange(ppw, dtype=jnp.int32) % 16), (_NW, ppw))
    pad_dst = jnp.broadcast_to(
        (jnp.arange(ppw, dtype=jnp.int32) * 53) % _N, (_NW, ppw))
    src2d = jnp.concatenate(
        [src.reshape(_NW, epw), pad_src], axis=1).reshape(_NCHUNK, _C)
    dst2d = jnp.concatenate(
        [dst.reshape(_NW, epw), pad_dst], axis=1).reshape(_NCHUNK, _C)
    ninf = jnp.float32(-jnp.inf)
    pad1 = jnp.concatenate(
        [jnp.zeros((_NTAB - _N, _IN), jnp.float32),
         jnp.full((_NTAB - _N, 16), ninf)], axis=1)
    pad2 = jnp.concatenate(
        [jnp.zeros((_NTAB - _N, _NC + 1), jnp.float32),
         jnp.full((_NTAB - _N, 1), ninf),
         jnp.zeros((_NTAB - _N, 6), jnp.float32)], axis=1)

    h1ext, adt1 = _tc_a(x, W1, as1, ad1)
    p1 = _sc_pass(src2d, dst2d, jnp.concatenate([h1ext, pad1]), adt1,
                  _W1EXT, _compute1)
    h2ext, adt2 = _tc_b(p1[0], p1[1], b1.reshape(1, _IN), W2, mask.T,
                        a_src2.reshape(_NC, 1), a_dst2.reshape(_NC, 1))
    p2 = _sc_pass(src2d, dst2d, jnp.concatenate([h2ext, pad2]), adt2,
                  _W2EXT, _compute2)
    return _tc_c(p2[0], p2[1], b2.reshape(1, _NC))


# speedup vs baseline: 128.4772x; 1.0879x over previous
import functools

import jax
import jax.numpy as jnp
from jax import lax
from jax.experimental import pallas as pl
from jax.experimental.pallas import tpu as pltpu
from jax.experimental.pallas import tpu_sc as plsc

_N = 10000
_E = 320000
_IN = 128
_HID = 16
_HEADS = 8
_NC = 40

_NSC = 2
_NTILE = 16
_NW = _NSC * _NTILE
_C = 80
_KPW = 125
_NCHUNK = _E // _C
_LASTSET = (_KPW - 1) % 2
_ROWS_PER_TILE = _N // _NTILE
_ZROWS = 25

_W1EXT = 144
_W2EXT = 48

_BLK = 1000


def _tc_a_body(x_ref, w1_ref, as_ref, ad_ref, h1ext_ref, adt_ref):
    h1 = jnp.dot(x_ref[...], w1_ref[...], preferred_element_type=jnp.float32)
    asrc = jnp.dot(h1, as_ref[...], preferred_element_type=jnp.float32)
    adst = jnp.dot(h1, ad_ref[...], preferred_element_type=jnp.float32)
    z8 = jnp.zeros_like(asrc)
    h1ext_ref[:, :_IN] = h1
    h1ext_ref[:, _IN:_W1EXT] = jnp.concatenate([asrc, z8], axis=1)
    adt_ref[...] = jnp.concatenate([adst, z8], axis=1)


def _tc_a(x, w1, as1, ad1):
    grid = (_N // _BLK,)
    return pl.pallas_call(
        _tc_a_body,
        grid=grid,
        in_specs=[
            pl.BlockSpec((_BLK, _IN), lambda i: (i, 0)),
            pl.BlockSpec((_IN, _IN), lambda i: (0, 0)),
            pl.BlockSpec((_IN, _HEADS), lambda i: (0, 0)),
            pl.BlockSpec((_IN, _HEADS), lambda i: (0, 0)),
        ],
        out_specs=[
            pl.BlockSpec((_BLK, _W1EXT), lambda i: (i, 0)),
            pl.BlockSpec((_BLK, 16), lambda i: (i, 0)),
        ],
        out_shape=[
            jax.ShapeDtypeStruct((_N, _W1EXT), jnp.float32),
            jax.ShapeDtypeStruct((_N, 16), jnp.float32),
        ],
    )(x, w1, as1, ad1)


def _tc_b_body(a0_ref, a1_ref, b1_ref, w2_ref, r_ref, a2s_ref, a2d_ref,
               h2ext_ref, adt2_ref):
    acc = a0_ref[...] + a1_ref[...]
    inv = 1.0 / (acc[:, _IN:_IN + _HEADS] + 1e-16)
    inv128 = jnp.dot(inv, r_ref[...], preferred_element_type=jnp.float32)
    h2in = jnp.maximum(acc[:, :_IN] * inv128 + b1_ref[...], 0.0)
    h2 = jnp.dot(h2in, w2_ref[...], preferred_element_type=jnp.float32)
    asrc2 = jnp.dot(h2, a2s_ref[...], preferred_element_type=jnp.float32)
    adst2 = jnp.dot(h2, a2d_ref[...], preferred_element_type=jnp.float32)
    ones = jnp.ones_like(asrc2)
    z6 = jnp.zeros((h2.shape[0], 6), jnp.float32)
    h2ext_ref[...] = jnp.concatenate([h2, ones, asrc2, z6], axis=1)
    adt2_ref[...] = jnp.broadcast_to(adst2, (h2.shape[0], 16))


def _tc_b(a0, a1, b1r, w2, r, a2s, a2d):
    grid = (_N // _BLK,)
    return pl.pallas_call(
        _tc_b_body,
        grid=grid,
        in_specs=[
            pl.BlockSpec((_BLK, _W1EXT), lambda i: (i, 0)),
            pl.BlockSpec((_BLK, _W1EXT), lambda i: (i, 0)),
            pl.BlockSpec((1, _IN), lambda i: (0, 0)),
            pl.BlockSpec((_IN, _NC), lambda i: (0, 0)),
            pl.BlockSpec((_HEADS, _IN), lambda i: (0, 0)),
            pl.BlockSpec((_NC, 1), lambda i: (0, 0)),
            pl.BlockSpec((_NC, 1), lambda i: (0, 0)),
        ],
        out_specs=[
            pl.BlockSpec((_BLK, _W2EXT), lambda i: (i, 0)),
            pl.BlockSpec((_BLK, 16), lambda i: (i, 0)),
        ],
        out_shape=[
            jax.ShapeDtypeStruct((_N, _W2EXT), jnp.float32),
            jax.ShapeDtypeStruct((_N, 16), jnp.float32),
        ],
    )(a0, a1, b1r, w2, r, a2s, a2d)


def _tc_c_body(a0_ref, a1_ref, b2_ref, out_ref):
    acc = a0_ref[...] + a1_ref[...]
    den = acc[:, _NC:_NC + 1] + 1e-16
    out_ref[...] = acc[:, :_NC] / den + b2_ref[...]


def _tc_c(a0, a1, b2r):
    grid = (_N // _BLK,)
    return pl.pallas_call(
        _tc_c_body,
        grid=grid,
        in_specs=[
            pl.BlockSpec((_BLK, _W2EXT), lambda i: (i, 0)),
            pl.BlockSpec((_BLK, _W2EXT), lambda i: (i, 0)),
            pl.BlockSpec((1, _NC), lambda i: (0, 0)),
        ],
        out_specs=pl.BlockSpec((_BLK, _NC), lambda i: (i, 0)),
        out_shape=jax.ShapeDtypeStruct((_N, _NC), jnp.float32),
    )(a0, a1, b2r)


def _sc_zero_acc(s, zbuf, acc, width):
    def zrow(r, carry):
        for j in range(width // 16):
            zbuf[r, pl.ds(j * 16, 16)] = jnp.zeros((16,), jnp.float32)
        return carry
    lax.fori_loop(0, _ZROWS, zrow, 0)

    def zcopy(t, carry):
        pltpu.sync_copy(
            zbuf, acc.at[pl.ds(s * _ROWS_PER_TILE + t * _ZROWS, _ZROWS)])
        return carry
    lax.fori_loop(0, _ROWS_PER_TILE // _ZROWS, zcopy, 0)


def _sc_drain(c, s, acc, out_hbm):
    sl = pl.ds(s * _ROWS_PER_TILE, _ROWS_PER_TILE)
    pltpu.sync_copy(acc.at[sl], out_hbm.at[c, sl])


def _compute1(rows, adv):
    @plsc.parallel_loop(0, _C, unroll=4)
    def ebody(e):
        al = rows[e, pl.ds(_IN, 16)] + adv[e, :]
        al = jnp.where(al >= 0.0, al, al * 0.2)
        exv = jnp.exp(al)
        rows[e, pl.ds(_IN, 16)] = exv
        for h in range(_HEADS):
            rows[e, pl.ds(h * _HID, _HID)] = (
                rows[e, pl.ds(h * _HID, _HID)] * exv[h])


def _compute2(rows, adv):
    @plsc.parallel_loop(0, _C, unroll=8)
    def ebody(e):
        v2 = rows[e, pl.ds(32, 16)]
        al = v2 + adv[e, :]
        al = jnp.where(al >= 0.0, al, al * 0.2)
        ex = jnp.exp(al)[9]
        rows[e, pl.ds(0, 16)] = rows[e, pl.ds(0, 16)] * ex
        rows[e, pl.ds(16, 16)] = rows[e, pl.ds(16, 16)] * ex
        rows[e, pl.ds(32, 16)] = v2 * ex


def _make_sc_body(width, compute_chunk):
    def body(src_hbm, dst_hbm, tab_hbm, adt_hbm, out_hbm,
             srcv0, srcv1, dstv0, dstv1, dsts0, dsts1,
             rows0, rows1, adv0, adv1, zbuf, acc,
             semis0, semis1, semid0, semid1,
             semr0, semr1, sema0, sema1, sems0, sems1):
        c = lax.axis_index("c")
        s = lax.axis_index("s")
        wid = s * _NSC + c
        srcv = (srcv0, srcv1)
        dstv = (dstv0, dstv1)
        dsts = (dsts0, dsts1)
        rows = (rows0, rows1)
        adv = (adv0, adv1)
        semis = (semis0, semis1)
        semid = (semid0, semid1)
        semr = (semr0, semr1)
        sema = (sema0, sema1)
        sems = (sems0, sems1)
        start = wid * _KPW

        def idx_start(j, b):
            pltpu.async_copy(src_hbm.at[start + j], srcv[b], semis[b])
            pltpu.async_copy(dst_hbm.at[start + j], dstv[b], semid[b])

        def idx_wait(j, b):
            pltpu.make_async_copy(
                src_hbm.at[start + j], srcv[b], semis[b]).wait()
            pltpu.make_async_copy(
                dst_hbm.at[start + j], dstv[b], semid[b]).wait()

        def gather_start(b):
            pltpu.async_copy(tab_hbm.at[srcv[b]], rows[b], semr[b])
            pltpu.async_copy(adt_hbm.at[dstv[b]], adv[b], sema[b])

        def gather_wait(b):
            pltpu.make_async_copy(tab_hbm.at[srcv[b]], rows[b],
                                  semr[b]).wait()
            pltpu.make_async_copy(adt_hbm.at[dstv[b]], adv[b],
                                  sema[b]).wait()

        def scatter_wait(b):
            pltpu.make_async_copy(rows[b], acc.at[dsts[b]], sems[b]).wait()

        idx_start(0, 0)
        idx_start(1, 1)
        _sc_zero_acc(s, zbuf, acc, width)
        idx_wait(0, 0)
        gather_start(0)
        plsc.subcore_barrier()

        def outer(k, carry):
            for b in range(2):
                j = 2 * k + b

                @pl.when(j < _KPW)
                def _():
                    q = 1 - b

                    @pl.when(j >= 1)
                    def _():
                        scatter_wait(q)

                    @pl.when(j + 1 < _KPW)
                    def _():
                        idx_wait(j + 1, q)
                        gather_start(q)

                    gather_wait(b)
                    for v in range(_C // 16):
                        dsts[b][pl.ds(v * 16, 16)] = (
                            dstv[b][pl.ds(v * 16, 16)])

                    @pl.when(j + 2 < _KPW)
                    def _():
                        idx_start(j + 2, b)

                    compute_chunk(rows[b], adv[b])
                    pltpu.async_copy(rows[b], acc.at[dsts[b]], sems[b],
                                     add=True)
            return carry

        lax.fori_loop(0, (_KPW + 1) // 2, outer, 0)
        scatter_wait(_LASTSET)
        plsc.subcore_barrier()
        _sc_drain(c, s, acc, out_hbm)

    return body


def _sc_pass(src2d, dst2d, tab, adt, width, compute_chunk):
    mesh = plsc.VectorSubcoreMesh(
        core_axis_name="c", subcore_axis_name="s",
        num_cores=_NSC, num_subcores=_NTILE)
    return pl.kernel(
        _make_sc_body(width, compute_chunk),
        compiler_params=pltpu.CompilerParams(use_tc_tiling_on_sc=False),
        out_type=jax.ShapeDtypeStruct((_NSC, _N, width), jnp.float32),
        mesh=mesh,
        scratch_types=(
            [pltpu.VMEM((_C,), jnp.int32) for _ in range(6)]
            + [
                pltpu.VMEM((_C, width), jnp.float32),
                pltpu.VMEM((_C, width), jnp.float32),
                pltpu.VMEM((_C, 16), jnp.float32),
                pltpu.VMEM((_C, 16), jnp.float32),
                pltpu.VMEM((_ZROWS, width), jnp.float32),
                pltpu.VMEM_SHARED((_N, width), jnp.float32),
            ]
            + [pltpu.SemaphoreType.DMA for _ in range(10)]
        ),
    )(src2d, dst2d, tab, adt)


@jax.jit
def kernel(x, edge_index, W1, a_src1, a_dst1, b1, W2, a_src2, a_dst2, b2):
    src = edge_index[0].astype(jnp.int32)
    dst = edge_index[1].astype(jnp.int32)

    mask = (jnp.arange(_IN)[:, None] // _HID
            == jnp.arange(_HEADS)[None, :]).astype(jnp.float32)
    as1 = mask * a_src1.reshape(_IN)[:, None]
    ad1 = mask * a_dst1.reshape(_IN)[:, None]

    src2d = src.reshape(_NCHUNK, _C)
    dst2d = dst.reshape(_NCHUNK, _C)

    h1ext, adt1 = _tc_a(x, W1, as1, ad1)
    p1 = _sc_pass(src2d, dst2d, h1ext, adt1, _W1EXT, _compute1)
    h2ext, adt2 = _tc_b(p1[0], p1[1], b1.reshape(1, _IN), W2, mask.T,
                        a_src2.reshape(_NC, 1), a_dst2.reshape(_NC, 1))
    p2 = _sc_pass(src2d, dst2d, h2ext, adt2, _W2EXT, _compute2)
    return _tc_c(p2[0], p2[1], b2.reshape(1, _NC))
